# baseline (speedup 1.0000x reference)
import functools

import jax
import jax.numpy as jnp
from jax import lax
from jax.experimental import pallas as pl
from jax.experimental.pallas import tpu as pltpu
from jax.experimental.pallas import tpu_sc as plsc

_NC = 2
_NS = 16
_NW = _NC * _NS
_L = 16

_B = 4096
_T = 200
_N = 64
_V = 1000000
_R = _B * _T
_RW = _R // _NW
_IDXW = 100
_CB = 2
_CR = _CB * _T
_IDX_ROWS = _CR // _IDXW
_CHUNKS = _RW // _CR
_GRP = _N // _L

_VMAIN = (_V // 128) * 128
_COLS = _VMAIN // 128
_CPW = _COLS // _NW
_CREM = _COLS - _CPW * _NW
_VPAD = _VMAIN + 128


def _tr_body(tokT_hbm, tail_hbm, out_hbm, in0, in1, ou0, ou1, tail_v,
             si0, si1, so0, so1):
    wid = lax.axis_index("s") * _NC + lax.axis_index("c")
    in_v = (in0, in1)
    ou_v = (ou0, ou1)
    si = (si0, si1)
    so = (so0, so1)
    base = wid * _CPW

    n_idx = [g * _L + lax.iota(jnp.int32, _L) for g in range(_GRP)]

    def col_src(j):
        return tokT_hbm.at[:, pl.ds(j * 128, 128)]

    def col_dst(j):
        return out_hbm.at[pl.ds(j * 64, 64)]

    def start_read(jj, b):
        pltpu.make_async_copy(col_src(base + jj), in_v[b], si[b]).start()

    def wait_read(jj, b):
        pltpu.make_async_copy(col_src(base + jj), in_v[b], si[b]).wait()

    def start_write(jj, b):
        pltpu.make_async_copy(ou_v[b], col_dst(base + jj), so[b]).start()

    def wait_write(jj, b):
        pltpu.make_async_copy(ou_v[b], col_dst(base + jj), so[b]).wait()

    def transpose(b):
        iv = in_v[b]
        ov = ou_v[b]

        def tr_row(r0, acc):
            for dr in range(4):
                r = r0 * 4 + dr
                c_idx = jnp.full((_L,), r, jnp.int32)
                rr = r0 * 2 + dr // 2
                cbase = (dr % 2) * _N
                for g in range(_GRP):
                    vals = plsc.load_gather(iv, [n_idx[g], c_idx])
                    ov[rr, pl.ds(cbase + g * _L, _L)] = vals
            return acc

        lax.fori_loop(0, 32, tr_row, 0)

    start_read(0, 0)

    def pair_body(p2, acc):
        for par in range(2):
            jj = p2 * 2 + par
            b = par
            o = 1 - par
            wait_read(jj, b)

            @pl.when(jj >= 2)
            def _():
                wait_write(jj - 2, b)

            @pl.when(jj + 1 < _CPW)
            def _():
                start_read(jj + 1, o)

            transpose(b)
            start_write(jj, b)
        return acc

    lax.fori_loop(0, _CPW // 2, pair_body, 0)
    wait_write(_CPW - 2, 0)
    wait_write(_CPW - 1, 1)

    @pl.when(wid < _CREM)
    def _():
        j = _NW * _CPW + wid
        pltpu.sync_copy(tokT_hbm.at[:, pl.ds(j * 128, 128)], in_v[0])
        transpose(0)
        pltpu.sync_copy(ou_v[0], out_hbm.at[pl.ds(j * 64, 64)])

    @pl.when(wid == _NW - 1)
    def _():
        pltpu.sync_copy(tail_hbm, tail_v)
        pltpu.sync_copy(tail_v, out_hbm.at[pl.ds(_VMAIN // 2, 32)])


def _emb_body(tok_hbm, idx_hbm, pos_hbm, out_hbm,
              idx0, idx1, rows0, rows1, pos_v,
              sidx0, sidx1, sg0, sg1, so0, so1):
    wid = lax.axis_index("s") * _NC + lax.axis_index("c")
    idx_v = (idx0, idx1)
    rows_v = (rows0, rows1)
    sidx = (sidx0, sidx1)
    sg = (sg0, sg1)
    so = (so0, so1)

    pltpu.sync_copy(pos_hbm.at[pl.ds(0, _T)], pos_v)

    def idx_slice(c):
        irow = wid * (_RW // _IDXW) + c * _IDX_ROWS
        return idx_hbm.at[pl.ds(irow, _IDX_ROWS)]

    def out_slice(c):
        bbase = (wid * _RW + c * _CR) // _T
        return out_hbm.at[pl.ds(bbase, _CB)]

    def start_idx(c, b):
        pltpu.make_async_copy(idx_slice(c), idx_v[b], sidx[b]).start()

    def wait_idx(c, b):
        pltpu.make_async_copy(idx_slice(c), idx_v[b], sidx[b]).wait()

    def start_gathers(b):
        for g in range(_IDX_ROWS):
            rep, off = divmod(g * _IDXW, _T)
            pltpu.make_async_copy(
                tok_hbm.at[idx_v[b].at[g]],
                rows_v[b].at[rep, pl.ds(off, _IDXW)],
                sg[b],
            ).start()

    def wait_gathers(b):
        for rep in range(_CB):
            pltpu.make_async_copy(
                tok_hbm.at[pl.ds(0, _T)], rows_v[b].at[rep], sg[b]
            ).wait()

    def start_out(c, b):
        pltpu.make_async_copy(rows_v[b], out_slice(c), so[b]).start()

    def wait_out(c, b):
        pltpu.make_async_copy(rows_v[b], out_slice(c), so[b]).wait()

    def add_pos(b):
        rv = rows_v[b]

        def add_body(r0, acc):
            for dr in range(4):
                r = r0 * 4 + dr
                for g in range(_GRP):
                    sl = pl.ds(g * _L, _L)
                    p = pos_v[r, sl]
                    for rep in range(_CB):
                        rv[rep, r, sl] = rv[rep, r, sl] + p
            return acc

        lax.fori_loop(0, _T // 4, add_body, 0, unroll=2)

    start_idx(0, 0)

    def pair_body(c2, acc):
        for par in range(2):
            c = c2 * 2 + par
            b = par
            o = 1 - par
            wait_idx(c, b)

            @pl.when(c >= 2)
            def _():
                wait_out(c - 2, b)

            start_gathers(b)

            @pl.when(c >= 1)
            def _():
                wait_gathers(o)

            @pl.when(c + 1 < _CHUNKS)
            def _():
                start_idx(c + 1, o)

            @pl.when(c >= 1)
            def _():
                add_pos(o)
                start_out(c - 1, o)
        return acc

    lax.fori_loop(0, _CHUNKS // 2, pair_body, 0)

    bl = (_CHUNKS - 1) % 2
    wait_gathers(bl)
    add_pos(bl)
    start_out(_CHUNKS - 1, bl)
    wait_out(_CHUNKS - 2, 1 - bl)
    wait_out(_CHUNKS - 1, bl)


@functools.partial(jax.jit, static_argnums=())
def kernel(idx, tok_emb, pos_emb):
    idx2d = idx.astype(jnp.int32).reshape(-1, _IDXW)
    tokT = jnp.transpose(tok_emb)
    tail = tok_emb[_VMAIN:].reshape(32, 128)
    mesh = plsc.VectorSubcoreMesh(core_axis_name="c", subcore_axis_name="s")

    tr = pl.kernel(
        _tr_body,
        out_type=jax.ShapeDtypeStruct((_VPAD // 2, 128), jnp.float32),
        mesh=mesh,
        scratch_types=[
            pltpu.VMEM((_N, 128), jnp.float32),
            pltpu.VMEM((_N, 128), jnp.float32),
            pltpu.VMEM((_N, 128), jnp.float32),
            pltpu.VMEM((_N, 128), jnp.float32),
            pltpu.VMEM((32, 128), jnp.float32),
            pltpu.SemaphoreType.DMA,
            pltpu.SemaphoreType.DMA,
            pltpu.SemaphoreType.DMA,
            pltpu.SemaphoreType.DMA,
        ],
        compiler_params=pltpu.CompilerParams(
            use_tc_tiling_on_sc=True, needs_layout_passes=False
        ),
    )
    tok_lin = tr(tokT, tail).reshape(_VPAD, _N)

    emb = pl.kernel(
        _emb_body,
        out_type=jax.ShapeDtypeStruct((_B, _T, _N), jnp.float32),
        mesh=mesh,
        scratch_types=[
            pltpu.VMEM((_IDX_ROWS, _IDXW), jnp.int32),
            pltpu.VMEM((_IDX_ROWS, _IDXW), jnp.int32),
            pltpu.VMEM((_CB, _T, _N), jnp.float32),
            pltpu.VMEM((_CB, _T, _N), jnp.float32),
            pltpu.VMEM((_T, _N), jnp.float32),
            pltpu.SemaphoreType.DMA,
            pltpu.SemaphoreType.DMA,
            pltpu.SemaphoreType.DMA,
            pltpu.SemaphoreType.DMA,
            pltpu.SemaphoreType.DMA,
            pltpu.SemaphoreType.DMA,
        ],
        compiler_params=pltpu.CompilerParams(use_tc_tiling_on_sc=False),
    )
    return emb(tok_lin, idx2d, pos_emb)


# speedup vs baseline: 1.4315x; 1.4315x over previous
import functools

import jax
import jax.numpy as jnp
from jax import lax
from jax.experimental import pallas as pl
from jax.experimental.pallas import tpu as pltpu
from jax.experimental.pallas import tpu_sc as plsc

_NC = 2
_NS = 16
_NW = _NC * _NS
_L = 16

_B = 4096
_T = 200
_N = 64
_V = 1000000
_R = _B * _T
_RW = _R // _NW
_IDXW = 100
_CB = 2
_CR = _CB * _T
_IDX_ROWS = _CR // _IDXW
_CHUNKS = _RW // _CR
_GRP = _N // _L

_VMAIN = (_V // 128) * 128
_BW = 256
_COLS = _VMAIN // _BW
_CPW = _COLS // _NW
_CREM = _COLS - _CPW * _NW
_VPAD = _VMAIN + 128


def _tr_body(tokT_hbm, tail_hbm, out_hbm, in0, in1, ou0, ou1, tail_v,
             si0, si1, so0, so1):
    wid = lax.axis_index("s") * _NC + lax.axis_index("c")
    in_v = (in0, in1)
    ou_v = (ou0, ou1)
    si = (si0, si1)
    so = (so0, so1)
    base = wid * _CPW

    n_idx = [g * _L + lax.iota(jnp.int32, _L) for g in range(_GRP)]

    def col_src(j):
        return tokT_hbm.at[:, pl.ds(j * _BW, _BW)]

    def col_dst(j):
        return out_hbm.at[pl.ds(j * (_BW // 2), _BW // 2)]

    def start_read(jj, b):
        pltpu.make_async_copy(col_src(base + jj), in_v[b], si[b]).start()

    def wait_read(jj, b):
        pltpu.make_async_copy(col_src(base + jj), in_v[b], si[b]).wait()

    def start_write(jj, b):
        pltpu.make_async_copy(ou_v[b], col_dst(base + jj), so[b]).start()

    def wait_write(jj, b):
        pltpu.make_async_copy(ou_v[b], col_dst(base + jj), so[b]).wait()

    def transpose(b):
        iv = in_v[b]
        ov = ou_v[b]

        @plsc.parallel_loop(0, _BW, unroll=8)
        def _(r):
            c_idx = jnp.full((_L,), r, jnp.int32)
            rr = lax.div(r, 2)
            cbase = lax.rem(r, 2) * _N
            for g in range(_GRP):
                vals = plsc.load_gather(iv, [n_idx[g], c_idx])
                ov[rr, pl.ds(cbase + g * _L, _L)] = vals

    start_read(0, 0)

    def pair_body(p2, acc):
        for par in range(2):
            jj = p2 * 2 + par
            b = par
            o = 1 - par
            wait_read(jj, b)

            @pl.when(jj >= 2)
            def _():
                wait_write(jj - 2, b)

            @pl.when(jj + 1 < _CPW)
            def _():
                start_read(jj + 1, o)

            transpose(b)
            start_write(jj, b)
        return acc

    lax.fori_loop(0, _CPW // 2, pair_body, 0)
    wait_write(_CPW - 2, 0)
    wait_write(_CPW - 1, 1)

    @pl.when(wid < _CREM)
    def _():
        j = _NW * _CPW + wid
        pltpu.sync_copy(tokT_hbm.at[:, pl.ds(j * _BW, _BW)], in_v[0])
        transpose(0)
        pltpu.sync_copy(ou_v[0], out_hbm.at[pl.ds(j * (_BW // 2), _BW // 2)])

    @pl.when(wid == _NW - 1)
    def _():
        pltpu.sync_copy(tail_hbm, tail_v)
        pltpu.sync_copy(tail_v, out_hbm.at[pl.ds(_VMAIN // 2, 32)])


def _emb_body(tok_hbm, idx_hbm, pos_hbm, out_hbm,
              idx0, idx1, rows0, rows1, pos_v,
              sidx0, sidx1, sg0, sg1, so0, so1):
    wid = lax.axis_index("s") * _NC + lax.axis_index("c")
    idx_v = (idx0, idx1)
    rows_v = (rows0, rows1)
    sidx = (sidx0, sidx1)
    sg = (sg0, sg1)
    so = (so0, so1)

    pltpu.sync_copy(pos_hbm.at[pl.ds(0, _T)], pos_v)

    def idx_slice(c):
        irow = wid * (_RW // _IDXW) + c * _IDX_ROWS
        return idx_hbm.at[pl.ds(irow, _IDX_ROWS)]

    def out_slice(c):
        bbase = (wid * _RW + c * _CR) // _T
        return out_hbm.at[pl.ds(bbase, _CB)]

    def start_idx(c, b):
        pltpu.make_async_copy(idx_slice(c), idx_v[b], sidx[b]).start()

    def wait_idx(c, b):
        pltpu.make_async_copy(idx_slice(c), idx_v[b], sidx[b]).wait()

    def start_gathers(b):
        for g in range(_IDX_ROWS):
            rep, off = divmod(g * _IDXW, _T)
            pltpu.make_async_copy(
                tok_hbm.at[idx_v[b].at[g]],
                rows_v[b].at[rep, pl.ds(off, _IDXW)],
                sg[b],
            ).start()

    def wait_gathers(b):
        for rep in range(_CB):
            pltpu.make_async_copy(
                tok_hbm.at[pl.ds(0, _T)], rows_v[b].at[rep], sg[b]
            ).wait()

    def start_out(c, b):
        pltpu.make_async_copy(rows_v[b], out_slice(c), so[b]).start()

    def wait_out(c, b):
        pltpu.make_async_copy(rows_v[b], out_slice(c), so[b]).wait()

    def add_pos(b):
        rv = rows_v[b]

        def add_body(r0, acc):
            for dr in range(4):
                r = r0 * 4 + dr
                for g in range(_GRP):
                    sl = pl.ds(g * _L, _L)
                    p = pos_v[r, sl]
                    for rep in range(_CB):
                        rv[rep, r, sl] = rv[rep, r, sl] + p
            return acc

        lax.fori_loop(0, _T // 4, add_body, 0, unroll=2)

    start_idx(0, 0)

    def pair_body(c2, acc):
        for par in range(2):
            c = c2 * 2 + par
            b = par
            o = 1 - par
            wait_idx(c, b)

            @pl.when(c >= 2)
            def _():
                wait_out(c - 2, b)

            start_gathers(b)

            @pl.when(c >= 1)
            def _():
                wait_gathers(o)

            @pl.when(c + 1 < _CHUNKS)
            def _():
                start_idx(c + 1, o)

            @pl.when(c >= 1)
            def _():
                add_pos(o)
                start_out(c - 1, o)
        return acc

    lax.fori_loop(0, _CHUNKS // 2, pair_body, 0)

    bl = (_CHUNKS - 1) % 2
    wait_gathers(bl)
    add_pos(bl)
    start_out(_CHUNKS - 1, bl)
    wait_out(_CHUNKS - 2, 1 - bl)
    wait_out(_CHUNKS - 1, bl)


@functools.partial(jax.jit, static_argnums=())
def kernel(idx, tok_emb, pos_emb):
    idx2d = idx.astype(jnp.int32).reshape(-1, _IDXW)
    tokT = jnp.transpose(tok_emb)
    tail = tok_emb[_VMAIN:].reshape(32, 128)
    mesh = plsc.VectorSubcoreMesh(core_axis_name="c", subcore_axis_name="s")

    tr = pl.kernel(
        _tr_body,
        out_type=jax.ShapeDtypeStruct((_VPAD // 2, 128), jnp.float32),
        mesh=mesh,
        scratch_types=[
            pltpu.VMEM((_N, _BW), jnp.float32),
            pltpu.VMEM((_N, _BW), jnp.float32),
            pltpu.VMEM((_BW // 2, 128), jnp.float32),
            pltpu.VMEM((_BW // 2, 128), jnp.float32),
            pltpu.VMEM((32, 128), jnp.float32),
            pltpu.SemaphoreType.DMA,
            pltpu.SemaphoreType.DMA,
            pltpu.SemaphoreType.DMA,
            pltpu.SemaphoreType.DMA,
        ],
        compiler_params=pltpu.CompilerParams(
            use_tc_tiling_on_sc=True, needs_layout_passes=False
        ),
    )
    tok_lin = tr(tokT, tail).reshape(_VPAD, _N)

    emb = pl.kernel(
        _emb_body,
        out_type=jax.ShapeDtypeStruct((_B, _T, _N), jnp.float32),
        mesh=mesh,
        scratch_types=[
            pltpu.VMEM((_IDX_ROWS, _IDXW), jnp.int32),
            pltpu.VMEM((_IDX_ROWS, _IDXW), jnp.int32),
            pltpu.VMEM((_CB, _T, _N), jnp.float32),
            pltpu.VMEM((_CB, _T, _N), jnp.float32),
            pltpu.VMEM((_T, _N), jnp.float32),
            pltpu.SemaphoreType.DMA,
            pltpu.SemaphoreType.DMA,
            pltpu.SemaphoreType.DMA,
            pltpu.SemaphoreType.DMA,
            pltpu.SemaphoreType.DMA,
            pltpu.SemaphoreType.DMA,
        ],
        compiler_params=pltpu.CompilerParams(use_tc_tiling_on_sc=False),
    )
    return emb(tok_lin, idx2d, pos_emb)


# speedup vs baseline: 1.4332x; 1.0012x over previous
import functools

import jax
import jax.numpy as jnp
from jax import lax
from jax.experimental import pallas as pl
from jax.experimental.pallas import tpu as pltpu
from jax.experimental.pallas import tpu_sc as plsc

_NC = 2
_NS = 16
_NW = _NC * _NS
_L = 16

_B = 4096
_T = 200
_N = 64
_V = 1000000
_R = _B * _T
_RW = _R // _NW
_IDXW = 100
_CB = 2
_CR = _CB * _T
_IDX_ROWS = _CR // _IDXW
_CHUNKS = _RW // _CR
_GRP = _N // _L

_VMAIN = (_V // 128) * 128
_BW = 256
_COLS = _VMAIN // _BW
_CPW = _COLS // _NW
_CREM = _COLS - _CPW * _NW
_VPAD = _VMAIN + 128


def _tr_body(tokT_hbm, tail_hbm, out_hbm, in0, in1, ou0, ou1, tail_v,
             si0, si1, so0, so1):
    wid = lax.axis_index("s") * _NC + lax.axis_index("c")
    in_v = (in0, in1)
    ou_v = (ou0, ou1)
    si = (si0, si1)
    so = (so0, so1)
    base = wid * _CPW

    n_idx = [g * _L + lax.iota(jnp.int32, _L) for g in range(_GRP)]

    def col_src(j):
        return tokT_hbm.at[:, pl.ds(j * _BW, _BW)]

    def col_dst(j):
        return out_hbm.at[pl.ds(j * (_BW // 2), _BW // 2)]

    def start_read(jj, b):
        pltpu.make_async_copy(
            col_src(base + jj), in_v[b].at[:, pl.ds(0, _BW)], si[b]
        ).start()

    def wait_read(jj, b):
        pltpu.make_async_copy(
            col_src(base + jj), in_v[b].at[:, pl.ds(0, _BW)], si[b]
        ).wait()

    def start_write(jj, b):
        pltpu.make_async_copy(ou_v[b], col_dst(base + jj), so[b]).start()

    def wait_write(jj, b):
        pltpu.make_async_copy(ou_v[b], col_dst(base + jj), so[b]).wait()

    def transpose(b):
        iv = in_v[b]
        ov = ou_v[b]

        @plsc.parallel_loop(0, _BW, unroll=8)
        def _(r):
            c_idx = jnp.full((_L,), r, jnp.int32)
            rr = lax.div(r, 2)
            cbase = lax.rem(r, 2) * _N
            for g in range(_GRP):
                vals = plsc.load_gather(iv, [n_idx[g], c_idx])
                ov[rr, pl.ds(cbase + g * _L, _L)] = vals

    start_read(0, 0)

    def pair_body(p2, acc):
        for par in range(2):
            jj = p2 * 2 + par
            b = par
            o = 1 - par
            wait_read(jj, b)

            @pl.when(jj >= 2)
            def _():
                wait_write(jj - 2, b)

            @pl.when(jj + 1 < _CPW)
            def _():
                start_read(jj + 1, o)

            transpose(b)
            start_write(jj, b)
        return acc

    lax.fori_loop(0, _CPW // 2, pair_body, 0)
    wait_write(_CPW - 2, 0)
    wait_write(_CPW - 1, 1)

    @pl.when(wid < _CREM)
    def _():
        j = _NW * _CPW + wid
        pltpu.sync_copy(
            tokT_hbm.at[:, pl.ds(j * _BW, _BW)], in_v[0].at[:, pl.ds(0, _BW)]
        )
        transpose(0)
        pltpu.sync_copy(ou_v[0], out_hbm.at[pl.ds(j * (_BW // 2), _BW // 2)])

    @pl.when(wid == _NW - 1)
    def _():
        pltpu.sync_copy(tail_hbm, tail_v)
        pltpu.sync_copy(tail_v, out_hbm.at[pl.ds(_VMAIN // 2, 32)])


def _emb_body(tok_hbm, idx_hbm, pos_hbm, out_hbm,
              idx0, idx1, rows0, rows1, pos_v,
              sidx0, sidx1, sg0, sg1, so0, so1):
    wid = lax.axis_index("s") * _NC + lax.axis_index("c")
    idx_v = (idx0, idx1)
    rows_v = (rows0, rows1)
    sidx = (sidx0, sidx1)
    sg = (sg0, sg1)
    so = (so0, so1)

    pltpu.sync_copy(pos_hbm.at[pl.ds(0, _T)], pos_v)

    def idx_slice(c):
        irow = wid * (_RW // _IDXW) + c * _IDX_ROWS
        return idx_hbm.at[pl.ds(irow, _IDX_ROWS)]

    def out_slice(c):
        bbase = (wid * _RW + c * _CR) // _T
        return out_hbm.at[pl.ds(bbase, _CB)]

    def start_idx(c, b):
        pltpu.make_async_copy(idx_slice(c), idx_v[b], sidx[b]).start()

    def wait_idx(c, b):
        pltpu.make_async_copy(idx_slice(c), idx_v[b], sidx[b]).wait()

    def start_gathers(b):
        for g in range(_IDX_ROWS):
            rep, off = divmod(g * _IDXW, _T)
            pltpu.make_async_copy(
                tok_hbm.at[idx_v[b].at[g]],
                rows_v[b].at[rep, pl.ds(off, _IDXW)],
                sg[b],
            ).start()

    def wait_gathers(b):
        for rep in range(_CB):
            pltpu.make_async_copy(
                tok_hbm.at[pl.ds(0, _T)], rows_v[b].at[rep], sg[b]
            ).wait()

    def start_out(c, b):
        pltpu.make_async_copy(rows_v[b], out_slice(c), so[b]).start()

    def wait_out(c, b):
        pltpu.make_async_copy(rows_v[b], out_slice(c), so[b]).wait()

    def add_pos(b):
        rv = rows_v[b]

        def add_body(r0, acc):
            for dr in range(4):
                r = r0 * 4 + dr
                for g in range(_GRP):
                    sl = pl.ds(g * _L, _L)
                    p = pos_v[r, sl]
                    for rep in range(_CB):
                        rv[rep, r, sl] = rv[rep, r, sl] + p
            return acc

        lax.fori_loop(0, _T // 4, add_body, 0, unroll=2)

    start_idx(0, 0)

    def pair_body(c2, acc):
        for par in range(2):
            c = c2 * 2 + par
            b = par
            o = 1 - par
            wait_idx(c, b)

            @pl.when(c >= 2)
            def _():
                wait_out(c - 2, b)

            start_gathers(b)

            @pl.when(c >= 1)
            def _():
                wait_gathers(o)

            @pl.when(c + 1 < _CHUNKS)
            def _():
                start_idx(c + 1, o)

            @pl.when(c >= 1)
            def _():
                add_pos(o)
                start_out(c - 1, o)
        return acc

    lax.fori_loop(0, _CHUNKS // 2, pair_body, 0)

    bl = (_CHUNKS - 1) % 2
    wait_gathers(bl)
    add_pos(bl)
    start_out(_CHUNKS - 1, bl)
    wait_out(_CHUNKS - 2, 1 - bl)
    wait_out(_CHUNKS - 1, bl)


@functools.partial(jax.jit, static_argnums=())
def kernel(idx, tok_emb, pos_emb):
    idx2d = idx.astype(jnp.int32).reshape(-1, _IDXW)
    tokT = jnp.transpose(tok_emb)
    tail = tok_emb[_VMAIN:].reshape(32, 128)
    mesh = plsc.VectorSubcoreMesh(core_axis_name="c", subcore_axis_name="s")

    tr = pl.kernel(
        _tr_body,
        out_type=jax.ShapeDtypeStruct((_VPAD // 2, 128), jnp.float32),
        mesh=mesh,
        scratch_types=[
            pltpu.VMEM((_N, _BW + 1), jnp.float32),
            pltpu.VMEM((_N, _BW + 1), jnp.float32),
            pltpu.VMEM((_BW // 2, 128), jnp.float32),
            pltpu.VMEM((_BW // 2, 128), jnp.float32),
            pltpu.VMEM((32, 128), jnp.float32),
            pltpu.SemaphoreType.DMA,
            pltpu.SemaphoreType.DMA,
            pltpu.SemaphoreType.DMA,
            pltpu.SemaphoreType.DMA,
        ],
        compiler_params=pltpu.CompilerParams(
            use_tc_tiling_on_sc=True, needs_layout_passes=False
        ),
    )
    tok_lin = tr(tokT, tail).reshape(_VPAD, _N)

    emb = pl.kernel(
        _emb_body,
        out_type=jax.ShapeDtypeStruct((_B, _T, _N), jnp.float32),
        mesh=mesh,
        scratch_types=[
            pltpu.VMEM((_IDX_ROWS, _IDXW), jnp.int32),
            pltpu.VMEM((_IDX_ROWS, _IDXW), jnp.int32),
            pltpu.VMEM((_CB, _T, _N), jnp.float32),
            pltpu.VMEM((_CB, _T, _N), jnp.float32),
            pltpu.VMEM((_T, _N), jnp.float32),
            pltpu.SemaphoreType.DMA,
            pltpu.SemaphoreType.DMA,
            pltpu.SemaphoreType.DMA,
            pltpu.SemaphoreType.DMA,
            pltpu.SemaphoreType.DMA,
            pltpu.SemaphoreType.DMA,
        ],
        compiler_params=pltpu.CompilerParams(use_tc_tiling_on_sc=False),
    )
    return emb(tok_lin, idx2d, pos_emb)


# speedup vs baseline: 1.6336x; 1.1399x over previous
import functools

import jax
import jax.numpy as jnp
from jax import lax
from jax.experimental import pallas as pl
from jax.experimental.pallas import tpu as pltpu
from jax.experimental.pallas import tpu_sc as plsc

_NC = 2
_NS = 16
_NW = _NC * _NS
_L = 16

_B = 4096
_T = 200
_N = 64
_R = _B * _T
_RW = _R // _NW
_IDXW = 100
_CB = 2
_CR = _CB * _T
_IDX_ROWS = _CR // _IDXW
_CHUNKS = _RW // _CR
_GRP = _N // _L
_UNROLL = 4


def _emb_body(tok_hbm, idx_hbm, pos_hbm, out_hbm,
              idx0, idx1, rows0, rows1, pos_v,
              sidx0, sidx1, sg0, sg1, so0, so1):
    wid = lax.axis_index("s") * _NC + lax.axis_index("c")
    idx_v = (idx0, idx1)
    rows_v = (rows0, rows1)
    sidx = (sidx0, sidx1)
    sg = (sg0, sg1)
    so = (so0, so1)

    pltpu.sync_copy(pos_hbm.at[pl.ds(0, _T)], pos_v)

    def idx_slice(c):
        irow = wid * (_RW // _IDXW) + c * _IDX_ROWS
        return idx_hbm.at[pl.ds(irow, _IDX_ROWS)]

    def out_slice(c):
        bbase = (wid * _RW + c * _CR) // _T
        return out_hbm.at[pl.ds(bbase, _CB)]

    def start_idx(c, b):
        pltpu.make_async_copy(idx_slice(c), idx_v[b], sidx[b]).start()

    def wait_idx(c, b):
        pltpu.make_async_copy(idx_slice(c), idx_v[b], sidx[b]).wait()

    def start_gathers(b):
        for g in range(_IDX_ROWS):
            rep, off = divmod(g * _IDXW, _T)
            pltpu.make_async_copy(
                tok_hbm.at[idx_v[b].at[g]],
                rows_v[b].at[rep, pl.ds(off, _IDXW)],
                sg[b],
            ).start()

    def wait_gathers(b):
        for rep in range(_CB):
            pltpu.make_async_copy(
                tok_hbm.at[pl.ds(0, _T)], rows_v[b].at[rep], sg[b]
            ).wait()

    def start_out(c, b):
        pltpu.make_async_copy(rows_v[b], out_slice(c), so[b]).start()

    def wait_out(c, b):
        pltpu.make_async_copy(rows_v[b], out_slice(c), so[b]).wait()

    def add_pos(b):
        rv = rows_v[b]

        def add_body(r0, acc):
            for dr in range(_UNROLL):
                r = r0 * _UNROLL + dr
                for g in range(_GRP):
                    sl = pl.ds(g * _L, _L)
                    p = pos_v[r, sl]
                    for rep in range(_CB):
                        rv[rep, r, sl] = rv[rep, r, sl] + p
            return acc

        lax.fori_loop(0, _T // _UNROLL, add_body, 0, unroll=2)

    start_idx(0, 0)

    def pair_body(c2, acc):
        for par in range(2):
            c = c2 * 2 + par
            b = par
            o = 1 - par
            wait_idx(c, b)

            @pl.when(c >= 2)
            def _():
                wait_out(c - 2, b)

            start_gathers(b)

            @pl.when(c >= 1)
            def _():
                wait_gathers(o)

            @pl.when(c + 1 < _CHUNKS)
            def _():
                start_idx(c + 1, o)

            @pl.when(c >= 1)
            def _():
                add_pos(o)
                start_out(c - 1, o)
        return acc

    lax.fori_loop(0, _CHUNKS // 2, pair_body, 0)

    bl = (_CHUNKS - 1) % 2
    wait_gathers(bl)
    add_pos(bl)
    start_out(_CHUNKS - 1, bl)
    wait_out(_CHUNKS - 2, 1 - bl)
    wait_out(_CHUNKS - 1, bl)


@functools.partial(jax.jit, static_argnums=())
def kernel(idx, tok_emb, pos_emb):
    b, t = idx.shape
    n = tok_emb.shape[1]
    idx2d = idx.astype(jnp.int32).reshape(-1, _IDXW)
    mesh = plsc.VectorSubcoreMesh(core_axis_name="c", subcore_axis_name="s")
    emb = pl.kernel(
        _emb_body,
        out_type=jax.ShapeDtypeStruct((_B, _T, _N), jnp.float32),
        mesh=mesh,
        scratch_types=[
            pltpu.VMEM((_IDX_ROWS, _IDXW), jnp.int32),
            pltpu.VMEM((_IDX_ROWS, _IDXW), jnp.int32),
            pltpu.VMEM((_CB, _T, _N), jnp.float32),
            pltpu.VMEM((_CB, _T, _N), jnp.float32),
            pltpu.VMEM((_T, _N), jnp.float32),
            pltpu.SemaphoreType.DMA,
            pltpu.SemaphoreType.DMA,
            pltpu.SemaphoreType.DMA,
            pltpu.SemaphoreType.DMA,
            pltpu.SemaphoreType.DMA,
            pltpu.SemaphoreType.DMA,
        ],
        compiler_params=pltpu.CompilerParams(use_tc_tiling_on_sc=False),
    )
    return emb(tok_emb, idx2d, pos_emb)
